# Initial kernel scaffold; baseline (speedup 1.0000x reference)
import jax
import jax.numpy as jnp
from jax.experimental import pallas as pl


def kernel(e, r, idx, et, Ws_w, Ws_b, Wn_w, Wn_b, Wr_w, Wr_b):
    raise NotImplementedError("write your pallas kernel here")


# speedup vs baseline: 3.4541x; 3.4541x over previous
import functools

import jax
import jax.numpy as jnp
from jax import lax
from jax.experimental import pallas as pl
from jax.experimental.pallas import tpu as pltpu
from jax.experimental.pallas import tpu_sc as plsc

N = 10000
E = 160000
R = 64
DIM = 256
H = DIM // 2
NC = 2
NS = 16
LANES = 16
EP = E // NS
K = 80
NCHUNK = EP // K
N_PAD = 10240
ZROWS = N_PAD // NS


NSTAGE = 25
GROUP = NSTAGE * K
NGROUP = EP // GROUP


def _sc_agg(e2, sig2, sidx, didx, et):
    mesh = plsc.VectorSubcoreMesh(core_axis_name="c", subcore_axis_name="s")

    @functools.partial(
        pl.kernel,
        mesh=mesh,
        out_type=jax.ShapeDtypeStruct((NC * N_PAD, H), jnp.float32),
        scratch_types=[
            pltpu.VMEM((GROUP,), jnp.int32),
            pltpu.VMEM((GROUP,), jnp.int32),
            pltpu.VMEM((GROUP,), jnp.int32),
            pltpu.VMEM((K,), jnp.int32),
            pltpu.VMEM((K,), jnp.int32),
            pltpu.VMEM((K,), jnp.int32),
            pltpu.VMEM((K, H), jnp.float32),
            pltpu.VMEM((K, H), jnp.float32),
            pltpu.VMEM_SHARED((N_PAD, H), jnp.float32),
        ],
    )
    def body(e2_hbm, sig2_hbm, sidx_hbm, didx_hbm, et_hbm, out_hbm,
             ss_v, st_v, sd_v, gs_v, gt_v, d_v, rows_v, sig_v, agg_sh):
        cid = lax.axis_index("c")
        sid = lax.axis_index("s")
        eoff = cid * N
        roff = cid * R

        @pl.loop(0, K)
        def _(i):
            for j in range(0, H, LANES):
                rows_v[i, pl.ds(j, LANES)] = jnp.zeros((LANES,), jnp.float32)

        row0 = sid * ZROWS
        assert ZROWS % K == 0
        for z in range(ZROWS // K):
            pltpu.sync_copy(rows_v, agg_sh.at[pl.ds(row0 + z * K, K)])
        plsc.subcore_barrier()

        @pl.loop(0, NGROUP)
        def _(g):
            gbase = sid * EP + g * GROUP
            pltpu.sync_copy(sidx_hbm.at[pl.ds(gbase, GROUP)], ss_v)
            pltpu.sync_copy(et_hbm.at[pl.ds(gbase, GROUP)], st_v)
            pltpu.sync_copy(didx_hbm.at[pl.ds(gbase, GROUP)], sd_v)

            @pl.loop(0, NSTAGE)
            def _(c):
                for i in range(0, K, LANES):
                    gs_v[pl.ds(i, LANES)] = ss_v[pl.ds(c * K + i, LANES)] + eoff
                    gt_v[pl.ds(i, LANES)] = st_v[pl.ds(c * K + i, LANES)] + roff
                    d_v[pl.ds(i, LANES)] = sd_v[pl.ds(c * K + i, LANES)]

                pltpu.sync_copy(e2_hbm.at[gs_v], rows_v)
                pltpu.sync_copy(sig2_hbm.at[gt_v], sig_v)

                @pl.loop(0, K)
                def _(i):
                    for j in range(0, H, LANES):
                        rows_v[i, pl.ds(j, LANES)] = (
                            rows_v[i, pl.ds(j, LANES)]
                            * sig_v[i, pl.ds(j, LANES)])

                pltpu.sync_copy(rows_v, agg_sh.at[d_v], add=True)

        plsc.subcore_barrier()
        pltpu.sync_copy(agg_sh.at[pl.ds(row0, ZROWS)],
                        out_hbm.at[pl.ds(cid * N_PAD + row0, ZROWS)])

    return body(e2, sig2, sidx, didx, et)


def _tc_rel(r, Wr_w, Wr_b):
    def body(r_ref, w_ref, b_ref, outr_ref, sig_ref):
        rr = r_ref[...]
        outr_ref[...] = lax.dot_general(
            rr, w_ref[...], (((1,), (1,)), ((), ())),
            preferred_element_type=jnp.float32) + b_ref[...]
        s = jax.nn.sigmoid(rr)
        sig_ref[0:R, :] = s[:, 0:H]
        sig_ref[R:2 * R, :] = s[:, H:DIM]

    return pl.pallas_call(
        body,
        out_shape=(jax.ShapeDtypeStruct((R, DIM), jnp.float32),
                   jax.ShapeDtypeStruct((2 * R, H), jnp.float32)),
    )(r, Wr_w, Wr_b)


BM = 2000


def _tc_out(e, agg3, Ws_w, Wn_w, Ws_b, Wn_b):
    def body(e_ref, a_ref, ws_ref, wn_ref, bs_ref, bn_ref, o_ref):
        x = lax.dot_general(e_ref[...], ws_ref[...], (((1,), (1,)), ((), ())),
                            preferred_element_type=jnp.float32)
        x = x + lax.dot_general(a_ref[0], wn_ref[:, 0:H],
                                (((1,), (1,)), ((), ())),
                                preferred_element_type=jnp.float32)
        x = x + lax.dot_general(a_ref[1], wn_ref[:, H:DIM],
                                (((1,), (1,)), ((), ())),
                                preferred_element_type=jnp.float32)
        o_ref[...] = jnp.maximum(x + bs_ref[...] + bn_ref[...], 0.0)

    return pl.pallas_call(
        body,
        grid=(N // BM,),
        in_specs=[
            pl.BlockSpec((BM, DIM), lambda i: (i, 0)),
            pl.BlockSpec((NC, BM, H), lambda i: (0, i, 0)),
            pl.BlockSpec((DIM, DIM), lambda i: (0, 0)),
            pl.BlockSpec((DIM, DIM), lambda i: (0, 0)),
            pl.BlockSpec((DIM,), lambda i: (0,)),
            pl.BlockSpec((DIM,), lambda i: (0,)),
        ],
        out_specs=pl.BlockSpec((BM, DIM), lambda i: (i, 0)),
        out_shape=jax.ShapeDtypeStruct((N, DIM), jnp.float32),
    )(e, agg3, Ws_w, Wn_w, Ws_b, Wn_b)


def kernel(e, r, idx, et, Ws_w, Ws_b, Wn_w, Wn_b, Wr_w, Wr_b):
    idx = idx.astype(jnp.int32)
    et32 = et.astype(jnp.int32)
    e2 = jnp.concatenate([e[:, :H], e[:, H:]], axis=0)

    out_r, sig2 = _tc_rel(r, Wr_w, Wr_b)
    agg2 = _sc_agg(e2, sig2, idx[0], idx[1], et32)
    agg3 = agg2.reshape(NC, N_PAD, H)
    out_e = _tc_out(e, agg3, Ws_w, Wn_w, Ws_b, Wn_b)
    return (out_e, out_r)
